# baseline (speedup 1.0000x reference)
import functools
import math

import jax
import jax.numpy as jnp
from jax import lax
from jax.experimental import pallas as pl
from jax.experimental.pallas import tpu as pltpu
from jax.experimental.pallas import tpu_sc as plsc

N = 10000
F = 128
H = 64
HH = H // 2
C = 2
E = 320000

NT = 16
CHUNK = 128
NBUF = 8
NCH = 160
EPT = NCH * CHUNK
EPAD = EPT * NT
NPAD = 10240
TROWS = NPAD // NT
INV_BN = float(1.0 / math.sqrt(1.0 + 1e-5))

RB = 1000


def _sc_agg(h0, h1, src, dst, zeros_tab):
    mesh = plsc.VectorSubcoreMesh(core_axis_name="c", subcore_axis_name="s")

    @functools.partial(
        pl.kernel,
        mesh=mesh,
        out_type=jax.ShapeDtypeStruct((2, NPAD, HH), jnp.float32),
        scratch_types=[
            pltpu.VMEM((NCH, CHUNK), jnp.int32),
            pltpu.VMEM((NCH, CHUNK), jnp.int32),
            [pltpu.VMEM((CHUNK, HH), jnp.float32) for _ in range(NBUF)],
            pltpu.VMEM_SHARED((NPAD, HH), jnp.float32),
            [pltpu.SemaphoreType.DMA for _ in range(NBUF)],
            [pltpu.SemaphoreType.DMA for _ in range(NBUF)],
        ],
        compiler_params=pltpu.CompilerParams(use_tc_tiling_on_sc=False),
    )
    def agg_kernel(h0_hbm, h1_hbm, src_hbm, dst_hbm, z_hbm, out_hbm,
                   src_v, dst_v, bufs, table_s, sems, ssems):
        cid = lax.axis_index("c")
        sid = lax.axis_index("s")
        r0 = sid * TROWS
        cp_s = pltpu.async_copy(src_hbm.at[sid], src_v, sems[0])
        cp_d = pltpu.async_copy(dst_hbm.at[sid], dst_v, sems[1])
        pltpu.sync_copy(z_hbm.at[pl.ds(r0, TROWS)], table_s.at[pl.ds(r0, TROWS)])
        cp_s.wait()
        cp_d.wait()
        plsc.subcore_barrier()

        def start_g(j, b):
            @pl.when(cid == 0)
            def _():
                pltpu.async_copy(h0_hbm.at[src_v.at[j]], bufs[b], sems[b])

            @pl.when(cid == 1)
            def _():
                pltpu.async_copy(h1_hbm.at[src_v.at[j]], bufs[b], sems[b])

        def wait_g(b):
            pltpu.make_async_copy(h0_hbm.at[pl.ds(0, CHUNK)],
                                  bufs[b], sems[b]).wait()

        def start_s(j, b):
            pltpu.async_copy(bufs[b], table_s.at[dst_v.at[j]], add=True,
                             sem=ssems[b])

        def wait_s(b):
            pltpu.make_async_copy(bufs[b], table_s.at[pl.ds(0, CHUNK)],
                                  ssems[b]).wait()

        for b in range(NBUF):
            start_g(b, b)

        def body(g, carry):
            j0 = g * NBUF
            for b in range(NBUF):
                wait_g(b)
                start_s(j0 + b, b)
            for b in range(NBUF):
                wait_s(b)
                start_g(j0 + b + NBUF, b)
            return carry

        lax.fori_loop(0, NCH // NBUF - 1, body, 0)
        j0 = NCH - NBUF
        for b in range(NBUF):
            wait_g(b)
            start_s(j0 + b, b)
        for b in range(NBUF):
            wait_s(b)

        plsc.subcore_barrier()
        pltpu.sync_copy(table_s.at[pl.ds(r0, TROWS)],
                        out_hbm.at[cid, pl.ds(r0, TROWS)])

    return agg_kernel(h0, h1, src, dst, zeros_tab)


def _embed(x, W_in, b_in):
    def body(x_ref, w_ref, b_ref, o0_ref, o1_ref):
        v = jnp.dot(x_ref[...], w_ref[...],
                    preferred_element_type=jnp.float32) + b_ref[...]
        v = jnp.maximum(v * INV_BN, 0.0)
        o0_ref[...] = v[:, :HH]
        o1_ref[...] = v[:, HH:]

    return pl.pallas_call(
        body,
        grid=(N // RB,),
        in_specs=[
            pl.BlockSpec((RB, F), lambda i: (i, 0)),
            pl.BlockSpec((F, H), lambda i: (0, 0)),
            pl.BlockSpec((1, H), lambda i: (0, 0)),
        ],
        out_specs=[pl.BlockSpec((RB, HH), lambda i: (i, 0)),
                   pl.BlockSpec((RB, HH), lambda i: (i, 0))],
        out_shape=[jax.ShapeDtypeStruct((N, HH), jnp.float32),
                   jax.ShapeDtypeStruct((N, HH), jnp.float32)],
    )(x, W_in, b_in.reshape(1, H))


def _gin_mlp(h0, h1, p, eps, Wa, ba, Wb, bb):
    def body(h0_ref, h1_ref, p_ref, e_ref, wa_ref, ba_ref, wb_ref, bb_ref,
             o0_ref, o1_ref):
        hb0 = h0_ref[...]
        hb1 = h1_ref[...]
        a = p_ref[...]
        scale = 1.0 + e_ref[0, 0]
        t = jnp.concatenate([hb0 * scale + a[0], hb1 * scale + a[1]],
                            axis=1)
        u = jnp.dot(t, wa_ref[...],
                    preferred_element_type=jnp.float32) + ba_ref[...]
        u = jnp.maximum(u * INV_BN, 0.0)
        v = jnp.dot(u, wb_ref[...],
                    preferred_element_type=jnp.float32) + bb_ref[...]
        v = v * INV_BN
        o0_ref[...] = v[:, :HH] + hb0
        o1_ref[...] = v[:, HH:] + hb1

    return pl.pallas_call(
        body,
        grid=(N // RB,),
        in_specs=[
            pl.BlockSpec((RB, HH), lambda i: (i, 0)),
            pl.BlockSpec((RB, HH), lambda i: (i, 0)),
            pl.BlockSpec((2, RB, HH), lambda i: (0, i, 0)),
            pl.BlockSpec((1, 1), lambda i: (0, 0)),
            pl.BlockSpec((H, H), lambda i: (0, 0)),
            pl.BlockSpec((1, H), lambda i: (0, 0)),
            pl.BlockSpec((H, H), lambda i: (0, 0)),
            pl.BlockSpec((1, H), lambda i: (0, 0)),
        ],
        out_specs=[pl.BlockSpec((RB, HH), lambda i: (i, 0)),
                   pl.BlockSpec((RB, HH), lambda i: (i, 0))],
        out_shape=[jax.ShapeDtypeStruct((N, HH), jnp.float32),
                   jax.ShapeDtypeStruct((N, HH), jnp.float32)],
    )(h0, h1, p, eps.reshape(1, 1), Wa, ba.reshape(1, H), Wb,
      bb.reshape(1, H))


def _gin_final(h0, h1, p, eps, Wa, ba, Wb, bb, Wc1, bc1, Wc2, bc2):
    G = N // RB

    def body(h0_ref, h1_ref, p_ref, e_ref, wa_ref, ba_ref, wb_ref, bb_ref,
             wc1_ref, bc1_ref, wc2_ref, bc2_ref, o_ref, acc_ref):
        i = pl.program_id(0)
        hb0 = h0_ref[...]
        hb1 = h1_ref[...]
        a = p_ref[...]
        scale = 1.0 + e_ref[0, 0]
        t = jnp.concatenate([hb0 * scale + a[0], hb1 * scale + a[1]],
                            axis=1)
        u = jnp.dot(t, wa_ref[...],
                    preferred_element_type=jnp.float32) + ba_ref[...]
        u = jnp.maximum(u * INV_BN, 0.0)
        v = jnp.dot(u, wb_ref[...],
                    preferred_element_type=jnp.float32) + bb_ref[...]
        v = v * INV_BN + jnp.concatenate([hb0, hb1], axis=1)
        s = jnp.sum(v, axis=0, keepdims=True)

        @pl.when(i == 0)
        def _():
            acc_ref[...] = s

        @pl.when(i > 0)
        def _():
            acc_ref[...] = acc_ref[...] + s

        @pl.when(i == G - 1)
        def _():
            pooled = acc_ref[...] * (1.0 / N)
            c1 = jnp.dot(pooled, wc1_ref[...],
                         preferred_element_type=jnp.float32) + bc1_ref[...]
            c1 = jnp.maximum(c1 * INV_BN, 0.0)
            o_ref[...] = jnp.dot(c1, wc2_ref[...],
                                 preferred_element_type=jnp.float32) + bc2_ref[...]

    return pl.pallas_call(
        body,
        grid=(G,),
        in_specs=[
            pl.BlockSpec((RB, HH), lambda i: (i, 0)),
            pl.BlockSpec((RB, HH), lambda i: (i, 0)),
            pl.BlockSpec((2, RB, HH), lambda i: (0, i, 0)),
            pl.BlockSpec((1, 1), lambda i: (0, 0)),
            pl.BlockSpec((H, H), lambda i: (0, 0)),
            pl.BlockSpec((1, H), lambda i: (0, 0)),
            pl.BlockSpec((H, H), lambda i: (0, 0)),
            pl.BlockSpec((1, H), lambda i: (0, 0)),
            pl.BlockSpec((H, HH), lambda i: (0, 0)),
            pl.BlockSpec((1, HH), lambda i: (0, 0)),
            pl.BlockSpec((HH, C), lambda i: (0, 0)),
            pl.BlockSpec((1, C), lambda i: (0, 0)),
        ],
        out_specs=pl.BlockSpec((1, C), lambda i: (0, 0)),
        out_shape=jax.ShapeDtypeStruct((1, C), jnp.float32),
        scratch_shapes=[pltpu.VMEM((1, H), jnp.float32)],
    )(h0, h1, p, eps.reshape(1, 1), Wa, ba.reshape(1, H), Wb,
      bb.reshape(1, H), Wc1, bc1.reshape(1, HH), Wc2, bc2.reshape(1, C))


def kernel(x, edge_index, W_in, b_in,
           eps1, W1a, b1a, W1b, b1b,
           eps2, W2a, b2a, W2b, b2b,
           eps3, W3a, b3a, W3b, b3b,
           Wc1, bc1, Wc2, bc2):
    ei = edge_index.astype(jnp.int32)
    pad = EPAD - E
    src = jnp.concatenate([ei[0], jnp.zeros((pad,), jnp.int32)])
    dst_pad = N + jnp.arange(pad, dtype=jnp.int32) % (NPAD - N)
    dst = jnp.concatenate([ei[1], dst_pad])
    src = src.reshape(NT, NCH, CHUNK)
    dst = dst.reshape(NT, NCH, CHUNK)
    zeros_tab = jnp.zeros((NPAD, HH), jnp.float32)

    h0, h1 = _embed(x, W_in, b_in)
    p = _sc_agg(h0, h1, src, dst, zeros_tab)
    h0, h1 = _gin_mlp(h0, h1, p, eps1, W1a, b1a, W1b, b1b)
    p = _sc_agg(h0, h1, src, dst, zeros_tab)
    h0, h1 = _gin_mlp(h0, h1, p, eps2, W2a, b2a, W2b, b2b)
    p = _sc_agg(h0, h1, src, dst, zeros_tab)
    return _gin_final(h0, h1, p, eps3, W3a, b3a, W3b, b3b, Wc1, bc1, Wc2, bc2)


# speedup vs baseline: 1.0470x; 1.0470x over previous
import functools
import math

import jax
import jax.numpy as jnp
from jax import lax
from jax.experimental import pallas as pl
from jax.experimental.pallas import tpu as pltpu
from jax.experimental.pallas import tpu_sc as plsc

N = 10000
F = 128
H = 64
HH = H // 2
C = 2
E = 320000

NT = 16
CHUNK = 128
NBUF = 8
NCH = 160
EPT = NCH * CHUNK
EPAD = EPT * NT
NPAD = 10240
TROWS = NPAD // NT
INV_BN = float(1.0 / math.sqrt(1.0 + 1e-5))

RB = 2000


def _sc_agg(h0, h1, src, dst, zeros_tab):
    mesh = plsc.VectorSubcoreMesh(core_axis_name="c", subcore_axis_name="s")

    @functools.partial(
        pl.kernel,
        mesh=mesh,
        out_type=jax.ShapeDtypeStruct((2, NPAD, HH), jnp.float32),
        scratch_types=[
            pltpu.VMEM((NCH, CHUNK), jnp.int32),
            pltpu.VMEM((NCH, CHUNK), jnp.int32),
            [pltpu.VMEM((CHUNK, HH), jnp.float32) for _ in range(NBUF)],
            pltpu.VMEM_SHARED((NPAD, HH), jnp.float32),
            [pltpu.SemaphoreType.DMA for _ in range(NBUF)],
        ],
        compiler_params=pltpu.CompilerParams(use_tc_tiling_on_sc=False),
    )
    def agg_kernel(h0_hbm, h1_hbm, src_hbm, dst_hbm, z_hbm, out_hbm,
                   src_v, dst_v, bufs, table_s, sems):
        cid = lax.axis_index("c")
        sid = lax.axis_index("s")
        r0 = sid * TROWS
        cp_s = pltpu.async_copy(src_hbm.at[sid], src_v, sems[0])
        cp_d = pltpu.async_copy(dst_hbm.at[sid], dst_v, sems[1])
        pltpu.sync_copy(z_hbm.at[pl.ds(r0, TROWS)], table_s.at[pl.ds(r0, TROWS)])
        cp_s.wait()
        cp_d.wait()
        plsc.subcore_barrier()

        def start_g(j, b):
            @pl.when(cid == 0)
            def _():
                pltpu.async_copy(h0_hbm.at[src_v.at[j]], bufs[b], sems[b])

            @pl.when(cid == 1)
            def _():
                pltpu.async_copy(h1_hbm.at[src_v.at[j]], bufs[b], sems[b])

        def wait_g(b):
            pltpu.make_async_copy(h0_hbm.at[pl.ds(0, CHUNK)],
                                  bufs[b], sems[b]).wait()

        def scat(j, b):
            pltpu.sync_copy(bufs[b], table_s.at[dst_v.at[j]], add=True)

        for b in range(NBUF):
            start_g(b, b)

        def body(g, carry):
            j0 = g * NBUF
            for b in range(NBUF):
                wait_g(b)
                scat(j0 + b, b)
                start_g(j0 + b + NBUF, b)
            return carry

        lax.fori_loop(0, NCH // NBUF - 1, body, 0)
        j0 = NCH - NBUF
        for b in range(NBUF):
            wait_g(b)
            scat(j0 + b, b)

        plsc.subcore_barrier()
        pltpu.sync_copy(table_s.at[pl.ds(r0, TROWS)],
                        out_hbm.at[cid, pl.ds(r0, TROWS)])

    return agg_kernel(h0, h1, src, dst, zeros_tab)


def _embed(x, W_in, b_in):
    def body(x_ref, w_ref, b_ref, o0_ref, o1_ref):
        v = jnp.dot(x_ref[...], w_ref[...],
                    preferred_element_type=jnp.float32) + b_ref[...]
        v = jnp.maximum(v * INV_BN, 0.0)
        o0_ref[...] = v[:, :HH]
        o1_ref[...] = v[:, HH:]

    return pl.pallas_call(
        body,
        grid=(N // RB,),
        in_specs=[
            pl.BlockSpec((RB, F), lambda i: (i, 0)),
            pl.BlockSpec((F, H), lambda i: (0, 0)),
            pl.BlockSpec((1, H), lambda i: (0, 0)),
        ],
        out_specs=[pl.BlockSpec((RB, HH), lambda i: (i, 0)),
                   pl.BlockSpec((RB, HH), lambda i: (i, 0))],
        out_shape=[jax.ShapeDtypeStruct((N, HH), jnp.float32),
                   jax.ShapeDtypeStruct((N, HH), jnp.float32)],
    )(x, W_in, b_in.reshape(1, H))


def _gin_mlp(h0, h1, p, eps, Wa, ba, Wb, bb):
    def body(h0_ref, h1_ref, p_ref, e_ref, wa_ref, ba_ref, wb_ref, bb_ref,
             o0_ref, o1_ref):
        hb0 = h0_ref[...]
        hb1 = h1_ref[...]
        a = p_ref[...]
        scale = 1.0 + e_ref[0, 0]
        t = jnp.concatenate([hb0 * scale + a[0], hb1 * scale + a[1]],
                            axis=1)
        u = jnp.dot(t, wa_ref[...],
                    preferred_element_type=jnp.float32) + ba_ref[...]
        u = jnp.maximum(u * INV_BN, 0.0)
        v = jnp.dot(u, wb_ref[...],
                    preferred_element_type=jnp.float32) + bb_ref[...]
        v = v * INV_BN
        o0_ref[...] = v[:, :HH] + hb0
        o1_ref[...] = v[:, HH:] + hb1

    return pl.pallas_call(
        body,
        grid=(N // RB,),
        in_specs=[
            pl.BlockSpec((RB, HH), lambda i: (i, 0)),
            pl.BlockSpec((RB, HH), lambda i: (i, 0)),
            pl.BlockSpec((2, RB, HH), lambda i: (0, i, 0)),
            pl.BlockSpec((1, 1), lambda i: (0, 0)),
            pl.BlockSpec((H, H), lambda i: (0, 0)),
            pl.BlockSpec((1, H), lambda i: (0, 0)),
            pl.BlockSpec((H, H), lambda i: (0, 0)),
            pl.BlockSpec((1, H), lambda i: (0, 0)),
        ],
        out_specs=[pl.BlockSpec((RB, HH), lambda i: (i, 0)),
                   pl.BlockSpec((RB, HH), lambda i: (i, 0))],
        out_shape=[jax.ShapeDtypeStruct((N, HH), jnp.float32),
                   jax.ShapeDtypeStruct((N, HH), jnp.float32)],
    )(h0, h1, p, eps.reshape(1, 1), Wa, ba.reshape(1, H), Wb,
      bb.reshape(1, H))


def _gin_final(h0, h1, p, eps, Wa, ba, Wb, bb, Wc1, bc1, Wc2, bc2):
    G = N // RB

    def body(h0_ref, h1_ref, p_ref, e_ref, wa_ref, ba_ref, wb_ref, bb_ref,
             wc1_ref, bc1_ref, wc2_ref, bc2_ref, o_ref, acc_ref):
        i = pl.program_id(0)
        hb0 = h0_ref[...]
        hb1 = h1_ref[...]
        a = p_ref[...]
        scale = 1.0 + e_ref[0, 0]
        t = jnp.concatenate([hb0 * scale + a[0], hb1 * scale + a[1]],
                            axis=1)
        u = jnp.dot(t, wa_ref[...],
                    preferred_element_type=jnp.float32) + ba_ref[...]
        u = jnp.maximum(u * INV_BN, 0.0)
        v = jnp.dot(u, wb_ref[...],
                    preferred_element_type=jnp.float32) + bb_ref[...]
        v = v * INV_BN + jnp.concatenate([hb0, hb1], axis=1)
        s = jnp.sum(v, axis=0, keepdims=True)

        @pl.when(i == 0)
        def _():
            acc_ref[...] = s

        @pl.when(i > 0)
        def _():
            acc_ref[...] = acc_ref[...] + s

        @pl.when(i == G - 1)
        def _():
            pooled = acc_ref[...] * (1.0 / N)
            c1 = jnp.dot(pooled, wc1_ref[...],
                         preferred_element_type=jnp.float32) + bc1_ref[...]
            c1 = jnp.maximum(c1 * INV_BN, 0.0)
            o_ref[...] = jnp.dot(c1, wc2_ref[...],
                                 preferred_element_type=jnp.float32) + bc2_ref[...]

    return pl.pallas_call(
        body,
        grid=(G,),
        in_specs=[
            pl.BlockSpec((RB, HH), lambda i: (i, 0)),
            pl.BlockSpec((RB, HH), lambda i: (i, 0)),
            pl.BlockSpec((2, RB, HH), lambda i: (0, i, 0)),
            pl.BlockSpec((1, 1), lambda i: (0, 0)),
            pl.BlockSpec((H, H), lambda i: (0, 0)),
            pl.BlockSpec((1, H), lambda i: (0, 0)),
            pl.BlockSpec((H, H), lambda i: (0, 0)),
            pl.BlockSpec((1, H), lambda i: (0, 0)),
            pl.BlockSpec((H, HH), lambda i: (0, 0)),
            pl.BlockSpec((1, HH), lambda i: (0, 0)),
            pl.BlockSpec((HH, C), lambda i: (0, 0)),
            pl.BlockSpec((1, C), lambda i: (0, 0)),
        ],
        out_specs=pl.BlockSpec((1, C), lambda i: (0, 0)),
        out_shape=jax.ShapeDtypeStruct((1, C), jnp.float32),
        scratch_shapes=[pltpu.VMEM((1, H), jnp.float32)],
    )(h0, h1, p, eps.reshape(1, 1), Wa, ba.reshape(1, H), Wb,
      bb.reshape(1, H), Wc1, bc1.reshape(1, HH), Wc2, bc2.reshape(1, C))


def kernel(x, edge_index, W_in, b_in,
           eps1, W1a, b1a, W1b, b1b,
           eps2, W2a, b2a, W2b, b2b,
           eps3, W3a, b3a, W3b, b3b,
           Wc1, bc1, Wc2, bc2):
    ei = edge_index.astype(jnp.int32)
    pad = EPAD - E
    src = jnp.concatenate([ei[0], jnp.zeros((pad,), jnp.int32)])
    dst_pad = N + jnp.arange(pad, dtype=jnp.int32) % (NPAD - N)
    dst = jnp.concatenate([ei[1], dst_pad])
    src = src.reshape(NT, NCH, CHUNK)
    dst = dst.reshape(NT, NCH, CHUNK)
    zeros_tab = jnp.zeros((NPAD, HH), jnp.float32)

    h0, h1 = _embed(x, W_in, b_in)
    p = _sc_agg(h0, h1, src, dst, zeros_tab)
    h0, h1 = _gin_mlp(h0, h1, p, eps1, W1a, b1a, W1b, b1b)
    p = _sc_agg(h0, h1, src, dst, zeros_tab)
    h0, h1 = _gin_mlp(h0, h1, p, eps2, W2a, b2a, W2b, b2b)
    p = _sc_agg(h0, h1, src, dst, zeros_tab)
    return _gin_final(h0, h1, p, eps3, W3a, b3a, W3b, b3b, Wc1, bc1, Wc2, bc2)


# speedup vs baseline: 1.6557x; 1.5813x over previous
import functools
import math

import jax
import jax.numpy as jnp
from jax import lax
from jax.experimental import pallas as pl
from jax.experimental.pallas import tpu as pltpu
from jax.experimental.pallas import tpu_sc as plsc

N = 10000
F = 128
H = 64
HH = H // 2
C = 2
E = 320000

NT = 16
CHUNK = 128
NBUF = 8
NCH = 160
EPT = NCH * CHUNK
EPAD = EPT * NT
NPAD = 10240
TROWS = NPAD // NT
INV_BN = float(1.0 / math.sqrt(1.0 + 1e-5))

RB = 2000


def _sc_agg(h0, h1, src, dst, zeros_tab):
    mesh = plsc.VectorSubcoreMesh(core_axis_name="c", subcore_axis_name="s")

    @functools.partial(
        pl.kernel,
        mesh=mesh,
        out_type=jax.ShapeDtypeStruct((2, NPAD, HH), jnp.bfloat16),
        scratch_types=[
            pltpu.VMEM((NCH, CHUNK), jnp.int32),
            pltpu.VMEM((NCH, CHUNK), jnp.int32),
            [pltpu.VMEM((CHUNK, HH), jnp.bfloat16) for _ in range(NBUF)],
            pltpu.VMEM_SHARED((NPAD, HH), jnp.bfloat16),
            [pltpu.SemaphoreType.DMA for _ in range(NBUF)],
        ],
        compiler_params=pltpu.CompilerParams(use_tc_tiling_on_sc=False),
    )
    def agg_kernel(h0_hbm, h1_hbm, src_hbm, dst_hbm, z_hbm, out_hbm,
                   src_v, dst_v, bufs, table_s, sems):
        cid = lax.axis_index("c")
        sid = lax.axis_index("s")
        r0 = sid * TROWS
        cp_s = pltpu.async_copy(src_hbm.at[sid], src_v, sems[0])
        cp_d = pltpu.async_copy(dst_hbm.at[sid], dst_v, sems[1])
        pltpu.sync_copy(z_hbm.at[pl.ds(r0, TROWS)], table_s.at[pl.ds(r0, TROWS)])
        cp_s.wait()
        cp_d.wait()
        plsc.subcore_barrier()

        def start_g(j, b):
            @pl.when(cid == 0)
            def _():
                pltpu.async_copy(h0_hbm.at[src_v.at[j]], bufs[b], sems[b])

            @pl.when(cid == 1)
            def _():
                pltpu.async_copy(h1_hbm.at[src_v.at[j]], bufs[b], sems[b])

        def wait_g(b):
            pltpu.make_async_copy(h0_hbm.at[pl.ds(0, CHUNK)],
                                  bufs[b], sems[b]).wait()

        def scat(j, b):
            pltpu.sync_copy(bufs[b], table_s.at[dst_v.at[j]], add=True)

        for b in range(NBUF):
            start_g(b, b)

        def body(g, carry):
            j0 = g * NBUF
            for b in range(NBUF):
                wait_g(b)
                scat(j0 + b, b)
                start_g(j0 + b + NBUF, b)
            return carry

        lax.fori_loop(0, NCH // NBUF - 1, body, 0)
        j0 = NCH - NBUF
        for b in range(NBUF):
            wait_g(b)
            scat(j0 + b, b)

        plsc.subcore_barrier()
        pltpu.sync_copy(table_s.at[pl.ds(r0, TROWS)],
                        out_hbm.at[cid, pl.ds(r0, TROWS)])

    return agg_kernel(h0, h1, src, dst, zeros_tab)


def _embed(x, W_in, b_in):
    def body(x_ref, w_ref, b_ref, o0_ref, o1_ref, g0_ref, g1_ref):
        v = jnp.dot(x_ref[...], w_ref[...],
                    preferred_element_type=jnp.float32) + b_ref[...]
        v = jnp.maximum(v * INV_BN, 0.0)
        o0_ref[...] = v[:, :HH]
        o1_ref[...] = v[:, HH:]
        g0_ref[...] = v[:, :HH].astype(jnp.bfloat16)
        g1_ref[...] = v[:, HH:].astype(jnp.bfloat16)

    return pl.pallas_call(
        body,
        grid=(N // RB,),
        in_specs=[
            pl.BlockSpec((RB, F), lambda i: (i, 0)),
            pl.BlockSpec((F, H), lambda i: (0, 0)),
            pl.BlockSpec((1, H), lambda i: (0, 0)),
        ],
        out_specs=[pl.BlockSpec((RB, HH), lambda i: (i, 0)),
                   pl.BlockSpec((RB, HH), lambda i: (i, 0)),
                   pl.BlockSpec((RB, HH), lambda i: (i, 0)),
                   pl.BlockSpec((RB, HH), lambda i: (i, 0))],
        out_shape=[jax.ShapeDtypeStruct((N, HH), jnp.float32),
                   jax.ShapeDtypeStruct((N, HH), jnp.float32),
                   jax.ShapeDtypeStruct((N, HH), jnp.bfloat16),
                   jax.ShapeDtypeStruct((N, HH), jnp.bfloat16)],
    )(x, W_in, b_in.reshape(1, H))


def _gin_mlp(h0, h1, p, eps, Wa, ba, Wb, bb):
    def body(h0_ref, h1_ref, p_ref, e_ref, wa_ref, ba_ref, wb_ref, bb_ref,
             o0_ref, o1_ref, g0_ref, g1_ref):
        hb0 = h0_ref[...]
        hb1 = h1_ref[...]
        a = p_ref[...].astype(jnp.float32)
        scale = 1.0 + e_ref[0, 0]
        t = jnp.concatenate([hb0 * scale + a[0], hb1 * scale + a[1]],
                            axis=1)
        u = jnp.dot(t, wa_ref[...],
                    preferred_element_type=jnp.float32) + ba_ref[...]
        u = jnp.maximum(u * INV_BN, 0.0)
        v = jnp.dot(u, wb_ref[...],
                    preferred_element_type=jnp.float32) + bb_ref[...]
        v = v * INV_BN
        n0 = v[:, :HH] + hb0
        n1 = v[:, HH:] + hb1
        o0_ref[...] = n0
        o1_ref[...] = n1
        g0_ref[...] = n0.astype(jnp.bfloat16)
        g1_ref[...] = n1.astype(jnp.bfloat16)

    return pl.pallas_call(
        body,
        grid=(N // RB,),
        in_specs=[
            pl.BlockSpec((RB, HH), lambda i: (i, 0)),
            pl.BlockSpec((RB, HH), lambda i: (i, 0)),
            pl.BlockSpec((2, RB, HH), lambda i: (0, i, 0)),
            pl.BlockSpec((1, 1), lambda i: (0, 0)),
            pl.BlockSpec((H, H), lambda i: (0, 0)),
            pl.BlockSpec((1, H), lambda i: (0, 0)),
            pl.BlockSpec((H, H), lambda i: (0, 0)),
            pl.BlockSpec((1, H), lambda i: (0, 0)),
        ],
        out_specs=[pl.BlockSpec((RB, HH), lambda i: (i, 0)),
                   pl.BlockSpec((RB, HH), lambda i: (i, 0)),
                   pl.BlockSpec((RB, HH), lambda i: (i, 0)),
                   pl.BlockSpec((RB, HH), lambda i: (i, 0))],
        out_shape=[jax.ShapeDtypeStruct((N, HH), jnp.float32),
                   jax.ShapeDtypeStruct((N, HH), jnp.float32),
                   jax.ShapeDtypeStruct((N, HH), jnp.bfloat16),
                   jax.ShapeDtypeStruct((N, HH), jnp.bfloat16)],
    )(h0, h1, p, eps.reshape(1, 1), Wa, ba.reshape(1, H), Wb,
      bb.reshape(1, H))


def _gin_final(h0, h1, p, eps, Wa, ba, Wb, bb, Wc1, bc1, Wc2, bc2):
    G = N // RB

    def body(h0_ref, h1_ref, p_ref, e_ref, wa_ref, ba_ref, wb_ref, bb_ref,
             wc1_ref, bc1_ref, wc2_ref, bc2_ref, o_ref, acc_ref):
        i = pl.program_id(0)
        hb0 = h0_ref[...]
        hb1 = h1_ref[...]
        a = p_ref[...].astype(jnp.float32)
        scale = 1.0 + e_ref[0, 0]
        t = jnp.concatenate([hb0 * scale + a[0], hb1 * scale + a[1]],
                            axis=1)
        u = jnp.dot(t, wa_ref[...],
                    preferred_element_type=jnp.float32) + ba_ref[...]
        u = jnp.maximum(u * INV_BN, 0.0)
        v = jnp.dot(u, wb_ref[...],
                    preferred_element_type=jnp.float32) + bb_ref[...]
        v = v * INV_BN + jnp.concatenate([hb0, hb1], axis=1)
        s = jnp.sum(v, axis=0, keepdims=True)

        @pl.when(i == 0)
        def _():
            acc_ref[...] = s

        @pl.when(i > 0)
        def _():
            acc_ref[...] = acc_ref[...] + s

        @pl.when(i == G - 1)
        def _():
            pooled = acc_ref[...] * (1.0 / N)
            c1 = jnp.dot(pooled, wc1_ref[...],
                         preferred_element_type=jnp.float32) + bc1_ref[...]
            c1 = jnp.maximum(c1 * INV_BN, 0.0)
            o_ref[...] = jnp.dot(c1, wc2_ref[...],
                                 preferred_element_type=jnp.float32) + bc2_ref[...]

    return pl.pallas_call(
        body,
        grid=(G,),
        in_specs=[
            pl.BlockSpec((RB, HH), lambda i: (i, 0)),
            pl.BlockSpec((RB, HH), lambda i: (i, 0)),
            pl.BlockSpec((2, RB, HH), lambda i: (0, i, 0)),
            pl.BlockSpec((1, 1), lambda i: (0, 0)),
            pl.BlockSpec((H, H), lambda i: (0, 0)),
            pl.BlockSpec((1, H), lambda i: (0, 0)),
            pl.BlockSpec((H, H), lambda i: (0, 0)),
            pl.BlockSpec((1, H), lambda i: (0, 0)),
            pl.BlockSpec((H, HH), lambda i: (0, 0)),
            pl.BlockSpec((1, HH), lambda i: (0, 0)),
            pl.BlockSpec((HH, C), lambda i: (0, 0)),
            pl.BlockSpec((1, C), lambda i: (0, 0)),
        ],
        out_specs=pl.BlockSpec((1, C), lambda i: (0, 0)),
        out_shape=jax.ShapeDtypeStruct((1, C), jnp.float32),
        scratch_shapes=[pltpu.VMEM((1, H), jnp.float32)],
    )(h0, h1, p, eps.reshape(1, 1), Wa, ba.reshape(1, H), Wb,
      bb.reshape(1, H), Wc1, bc1.reshape(1, HH), Wc2, bc2.reshape(1, C))


def kernel(x, edge_index, W_in, b_in,
           eps1, W1a, b1a, W1b, b1b,
           eps2, W2a, b2a, W2b, b2b,
           eps3, W3a, b3a, W3b, b3b,
           Wc1, bc1, Wc2, bc2):
    ei = edge_index.astype(jnp.int32)
    pad = EPAD - E
    src = jnp.concatenate([ei[0], jnp.zeros((pad,), jnp.int32)])
    dst_pad = N + jnp.arange(pad, dtype=jnp.int32) % (NPAD - N)
    dst = jnp.concatenate([ei[1], dst_pad])
    src = src.reshape(NT, NCH, CHUNK)
    dst = dst.reshape(NT, NCH, CHUNK)
    zeros_tab = jnp.zeros((NPAD, HH), jnp.bfloat16)

    h0, h1, g0, g1 = _embed(x, W_in, b_in)
    p = _sc_agg(g0, g1, src, dst, zeros_tab)
    h0, h1, g0, g1 = _gin_mlp(h0, h1, p, eps1, W1a, b1a, W1b, b1b)
    p = _sc_agg(g0, g1, src, dst, zeros_tab)
    h0, h1, g0, g1 = _gin_mlp(h0, h1, p, eps2, W2a, b2a, W2b, b2b)
    p = _sc_agg(g0, g1, src, dst, zeros_tab)
    return _gin_final(h0, h1, p, eps3, W3a, b3a, W3b, b3b, Wc1, bc1, Wc2, bc2)
